# baseline (speedup 1.0000x reference)
import functools

import jax
import jax.numpy as jnp
from jax import lax
from jax.experimental import pallas as pl
from jax.experimental.pallas import tpu as pltpu
from jax.experimental.pallas import tpu_sc as plsc

_B = 320000
_D = 128
_W = _D // 2
_L = 16
_NC = 2
_NS = 16
_NW = _NC * _NS
_PER_W = _B // _NW
_C = 80
_NCHUNK = _PER_W // _C
_NBUF = 8


def _tec_body(z_hbm, e0_hbm, e1_hbm, out_hbm,
              idx0_f, idx1_f, rows0_bufs, rows1_bufs, res_f,
              part_v, sems0, sems1):
    wid = lax.axis_index("s") * _NC + lax.axis_index("c")
    base = wid * _PER_W

    pltpu.sync_copy(e0_hbm.at[pl.ds(base, _PER_W)], idx0_f)
    pltpu.sync_copy(e1_hbm.at[pl.ds(base, _PER_W)], idx1_f)

    lanes = lax.iota(jnp.int32, _L)
    lanes17 = lanes * 17

    def issue(i, r0, r1, s0, s1):
        off = i * _C
        pltpu.async_copy(z_hbm.at[idx0_f.at[pl.ds(off, _C)]], r0, s0)
        pltpu.async_copy(z_hbm.at[idx1_f.at[pl.ds(off, _C)]], r1, s1)

    def wait(i, r0, r1, s0, s1):
        off = i * _C
        pltpu.make_async_copy(z_hbm.at[idx0_f.at[pl.ds(off, _C)]], r0, s0).wait()
        pltpu.make_async_copy(z_hbm.at[idx1_f.at[pl.ds(off, _C)]], r1, s1).wait()

    def compute(i, r0, r1):
        def group(g, _):
            base_c = g * _L
            for e2 in range(_L):
                c = base_c + e2
                s = []
                for l in range(_W // _L):
                    a = plsc.bitcast(r0[c, pl.ds(l * _L, _L)], jnp.bfloat16)
                    b = plsc.bitcast(r1[c, pl.ds(l * _L, _L)], jnp.bfloat16)
                    ae, ao = plsc.unpack(a, format=plsc.PackFormat.INTERLEAVED)
                    be, bo = plsc.unpack(b, format=plsc.PackFormat.INTERLEAVED)
                    s.append(ae * be)
                    s.append(ao * bo)
                acc = (((s[0] + s[1]) + (s[2] + s[3]))
                       + ((s[4] + s[5]) + (s[6] + s[7])))
                part_v[pl.ds(e2 * 17, _L)] = acc
            t = [plsc.load_gather(part_v, [lanes17 + l]) for l in range(_L)]
            t = [t[2 * k] + t[2 * k + 1] for k in range(8)]
            t = [t[2 * k] + t[2 * k + 1] for k in range(4)]
            tot = (t[0] + t[1]) + (t[2] + t[3])
            res_f[pl.ds(i * _C + base_c, _L)] = 1.0 / (1.0 + jnp.exp(-tot))
            return ()

        lax.fori_loop(0, _C // _L, group, ())

    for r in range(_NBUF):
        issue(r, rows0_bufs[r], rows1_bufs[r], sems0[r], sems1[r])

    def body(j, _):
        for r in range(_NBUF):
            i = _NBUF * j + r
            wait(i, rows0_bufs[r], rows1_bufs[r], sems0[r], sems1[r])
            compute(i, rows0_bufs[r], rows1_bufs[r])

            @pl.when(i + _NBUF < _NCHUNK)
            def _():
                issue(i + _NBUF, rows0_bufs[r], rows1_bufs[r],
                      sems0[r], sems1[r])

        return ()

    lax.fori_loop(0, _NCHUNK // _NBUF, body, ())

    for r in range(_NCHUNK % _NBUF):
        i = (_NCHUNK // _NBUF) * _NBUF + r
        wait(i, rows0_bufs[r], rows1_bufs[r], sems0[r], sems1[r])
        compute(i, rows0_bufs[r], rows1_bufs[r])

    pltpu.sync_copy(res_f, out_hbm.at[pl.ds(base, _PER_W)])


@functools.partial(jax.jit, static_argnums=())
def _sc_call(zp, e0, e1):
    mesh = plsc.VectorSubcoreMesh(core_axis_name="c", subcore_axis_name="s")
    f = pl.kernel(
        _tec_body,
        mesh=mesh,
        compiler_params=pltpu.CompilerParams(
            needs_layout_passes=False, use_tc_tiling_on_sc=False),
        out_type=jax.ShapeDtypeStruct((_B,), jnp.float32),
        scratch_types=[
            pltpu.VMEM((_PER_W,), jnp.int32),
            pltpu.VMEM((_PER_W,), jnp.int32),
            [pltpu.VMEM((_C, _W), jnp.int32) for _ in range(_NBUF)],
            [pltpu.VMEM((_C, _W), jnp.int32) for _ in range(_NBUF)],
            pltpu.VMEM((_PER_W,), jnp.float32),
            pltpu.VMEM((_L * 17,), jnp.float32),
            [pltpu.SemaphoreType.DMA for _ in range(_NBUF)],
            [pltpu.SemaphoreType.DMA for _ in range(_NBUF)],
        ],
    )
    return f(zp, e0, e1)


def kernel(z, e):
    zp = lax.bitcast_convert_type(
        z.astype(jnp.bfloat16).reshape(10000, _W, 2), jnp.int32)
    e0 = e[0].astype(jnp.int32)
    e1 = e[1].astype(jnp.int32)
    return _sc_call(zp, e0, e1)


# speedup vs baseline: 1.4470x; 1.4470x over previous
import functools

import jax
import jax.numpy as jnp
from jax import lax
from jax.experimental import pallas as pl
from jax.experimental.pallas import tpu as pltpu
from jax.experimental.pallas import tpu_sc as plsc

_B = 320000
_D = 128
_W = _D // 2
_L = 16
_NC = 2
_NS = 16
_NW = _NC * _NS
_PER_W = _B // _NW
_C = 80
_NCHUNK = _PER_W // _C
_NBUF = 2


def _tec_body(z_hbm, e0_hbm, e1_hbm, out_hbm,
              idx0_f, idx1_f, rows0_bufs, rows1_bufs, res_f,
              part_v, sems0, sems1):
    wid = lax.axis_index("s") * _NC + lax.axis_index("c")
    base = wid * _PER_W

    pltpu.sync_copy(e0_hbm.at[pl.ds(base, _PER_W)], idx0_f)
    pltpu.sync_copy(e1_hbm.at[pl.ds(base, _PER_W)], idx1_f)

    lanes = lax.iota(jnp.int32, _L)
    lanes17 = lanes * 17

    def issue(i, r0, r1, s0, s1):
        off = i * _C
        pltpu.async_copy(z_hbm.at[idx0_f.at[pl.ds(off, _C)]], r0, s0)
        pltpu.async_copy(z_hbm.at[idx1_f.at[pl.ds(off, _C)]], r1, s1)

    def wait(i, r0, r1, s0, s1):
        off = i * _C
        pltpu.make_async_copy(z_hbm.at[idx0_f.at[pl.ds(off, _C)]], r0, s0).wait()
        pltpu.make_async_copy(z_hbm.at[idx1_f.at[pl.ds(off, _C)]], r1, s1).wait()

    def compute(i, r0, r1):
        def group(g, _):
            base_c = g * _L
            accs = []
            for e2 in range(_L):
                c = base_c + e2
                s = []
                for l in range(_W // _L):
                    a = plsc.bitcast(r0[c, pl.ds(l * _L, _L)], jnp.bfloat16)
                    b = plsc.bitcast(r1[c, pl.ds(l * _L, _L)], jnp.bfloat16)
                    pe, po = plsc.unpack(a * b,
                                         format=plsc.PackFormat.INTERLEAVED)
                    s.append(pe)
                    s.append(po)
                accs.append((((s[0] + s[1]) + (s[2] + s[3]))
                             + ((s[4] + s[5]) + (s[6] + s[7]))))
            for e2 in range(_L):
                part_v[pl.ds(e2 * 17, _L)] = accs[e2]
            t = [plsc.load_gather(part_v, [lanes17 + l]) for l in range(_L)]
            t = [t[2 * k] + t[2 * k + 1] for k in range(8)]
            t = [t[2 * k] + t[2 * k + 1] for k in range(4)]
            tot = (t[0] + t[1]) + (t[2] + t[3])
            res_f[pl.ds(i * _C + base_c, _L)] = 1.0 / (1.0 + jnp.exp(-tot))
            return ()

        lax.fori_loop(0, _C // _L, group, ())

    for r in range(_NBUF):
        issue(r, rows0_bufs[r], rows1_bufs[r], sems0[r], sems1[r])

    def body(j, _):
        for r in range(_NBUF):
            i = _NBUF * j + r
            wait(i, rows0_bufs[r], rows1_bufs[r], sems0[r], sems1[r])
            compute(i, rows0_bufs[r], rows1_bufs[r])

            @pl.when(i + _NBUF < _NCHUNK)
            def _():
                issue(i + _NBUF, rows0_bufs[r], rows1_bufs[r],
                      sems0[r], sems1[r])

        return ()

    lax.fori_loop(0, _NCHUNK // _NBUF, body, ())

    for r in range(_NCHUNK % _NBUF):
        i = (_NCHUNK // _NBUF) * _NBUF + r
        wait(i, rows0_bufs[r], rows1_bufs[r], sems0[r], sems1[r])
        compute(i, rows0_bufs[r], rows1_bufs[r])

    pltpu.sync_copy(res_f, out_hbm.at[pl.ds(base, _PER_W)])


@functools.partial(jax.jit, static_argnums=())
def _sc_call(zp, e0, e1):
    mesh = plsc.VectorSubcoreMesh(core_axis_name="c", subcore_axis_name="s")
    f = pl.kernel(
        _tec_body,
        mesh=mesh,
        compiler_params=pltpu.CompilerParams(
            needs_layout_passes=False, use_tc_tiling_on_sc=False),
        out_type=jax.ShapeDtypeStruct((_B,), jnp.float32),
        scratch_types=[
            pltpu.VMEM((_PER_W,), jnp.int32),
            pltpu.VMEM((_PER_W,), jnp.int32),
            [pltpu.VMEM((_C, _W), jnp.int32) for _ in range(_NBUF)],
            [pltpu.VMEM((_C, _W), jnp.int32) for _ in range(_NBUF)],
            pltpu.VMEM((_PER_W,), jnp.float32),
            pltpu.VMEM((_L * 17,), jnp.float32),
            [pltpu.SemaphoreType.DMA for _ in range(_NBUF)],
            [pltpu.SemaphoreType.DMA for _ in range(_NBUF)],
        ],
    )
    return f(zp, e0, e1)


def kernel(z, e):
    zp = lax.bitcast_convert_type(
        z.astype(jnp.bfloat16).reshape(10000, _W, 2), jnp.int32)
    e0 = e[0].astype(jnp.int32)
    e1 = e[1].astype(jnp.int32)
    return _sc_call(zp, e0, e1)


# speedup vs baseline: 1.6807x; 1.1615x over previous
import functools

import jax
import jax.numpy as jnp
from jax import lax
from jax.experimental import pallas as pl
from jax.experimental.pallas import tpu as pltpu
from jax.experimental.pallas import tpu_sc as plsc

_B = 320000
_D = 128
_W = _D // 2
_L = 16
_NC = 2
_NS = 16
_NW = _NC * _NS
_PER_W = _B // _NW
_C = 80
_NCHUNK = _PER_W // _C
_NBUF = 4


def _tec_body(z_hbm, e0_hbm, e1_hbm, out_hbm,
              idx0_f, idx1_f, rows0_bufs, rows1_bufs, res_f,
              part_v, sems0, sems1):
    wid = lax.axis_index("s") * _NC + lax.axis_index("c")
    base = wid * _PER_W

    pltpu.sync_copy(e0_hbm.at[pl.ds(base, _PER_W)], idx0_f)
    pltpu.sync_copy(e1_hbm.at[pl.ds(base, _PER_W)], idx1_f)

    lanes = lax.iota(jnp.int32, _L)
    lanes17 = lanes * 17

    def issue(i, r0, r1, s0, s1):
        off = i * _C
        pltpu.async_copy(z_hbm.at[idx0_f.at[pl.ds(off, _C)]], r0, s0)
        pltpu.async_copy(z_hbm.at[idx1_f.at[pl.ds(off, _C)]], r1, s1)

    def wait(i, r0, r1, s0, s1):
        off = i * _C
        pltpu.make_async_copy(z_hbm.at[idx0_f.at[pl.ds(off, _C)]], r0, s0).wait()
        pltpu.make_async_copy(z_hbm.at[idx1_f.at[pl.ds(off, _C)]], r1, s1).wait()

    def compute(i, r0, r1):
        def group(g, _):
            base_c = g * _L
            accs = []
            for e2 in range(_L):
                c = base_c + e2
                s = []
                for l in range(_W // _L):
                    a = plsc.bitcast(r0[c, pl.ds(l * _L, _L)], jnp.bfloat16)
                    b = plsc.bitcast(r1[c, pl.ds(l * _L, _L)], jnp.bfloat16)
                    pe, po = plsc.unpack(a * b,
                                         format=plsc.PackFormat.INTERLEAVED)
                    s.append(pe)
                    s.append(po)
                accs.append((((s[0] + s[1]) + (s[2] + s[3]))
                             + ((s[4] + s[5]) + (s[6] + s[7]))))
            for e2 in range(_L):
                part_v[pl.ds(e2 * 17, _L)] = accs[e2]
            t = [plsc.load_gather(part_v, [lanes17 + l]) for l in range(_L)]
            t = [t[2 * k] + t[2 * k + 1] for k in range(8)]
            t = [t[2 * k] + t[2 * k + 1] for k in range(4)]
            tot = (t[0] + t[1]) + (t[2] + t[3])
            res_f[pl.ds(i * _C + base_c, _L)] = 1.0 / (1.0 + jnp.exp(-tot))
            return ()

        lax.fori_loop(0, _C // _L, group, ())

    for r in range(_NBUF):
        issue(r, rows0_bufs[r], rows1_bufs[r], sems0[r], sems1[r])

    def body(j, _):
        for r in range(_NBUF):
            i = _NBUF * j + r
            wait(i, rows0_bufs[r], rows1_bufs[r], sems0[r], sems1[r])
            compute(i, rows0_bufs[r], rows1_bufs[r])

            @pl.when(i + _NBUF < _NCHUNK)
            def _():
                issue(i + _NBUF, rows0_bufs[r], rows1_bufs[r],
                      sems0[r], sems1[r])

        return ()

    lax.fori_loop(0, _NCHUNK // _NBUF, body, ())

    for r in range(_NCHUNK % _NBUF):
        i = (_NCHUNK // _NBUF) * _NBUF + r
        wait(i, rows0_bufs[r], rows1_bufs[r], sems0[r], sems1[r])
        compute(i, rows0_bufs[r], rows1_bufs[r])

    pltpu.sync_copy(res_f, out_hbm.at[pl.ds(base, _PER_W)])


@functools.partial(jax.jit, static_argnums=())
def _sc_call(zp, e0, e1):
    mesh = plsc.VectorSubcoreMesh(core_axis_name="c", subcore_axis_name="s")
    f = pl.kernel(
        _tec_body,
        mesh=mesh,
        compiler_params=pltpu.CompilerParams(
            needs_layout_passes=False, use_tc_tiling_on_sc=False),
        out_type=jax.ShapeDtypeStruct((_B,), jnp.float32),
        scratch_types=[
            pltpu.VMEM((_PER_W,), jnp.int32),
            pltpu.VMEM((_PER_W,), jnp.int32),
            [pltpu.VMEM((_C, _W), jnp.int32) for _ in range(_NBUF)],
            [pltpu.VMEM((_C, _W), jnp.int32) for _ in range(_NBUF)],
            pltpu.VMEM((_PER_W,), jnp.float32),
            pltpu.VMEM((_L * 17,), jnp.float32),
            [pltpu.SemaphoreType.DMA for _ in range(_NBUF)],
            [pltpu.SemaphoreType.DMA for _ in range(_NBUF)],
        ],
    )
    return f(zp, e0, e1)


def kernel(z, e):
    zp = lax.bitcast_convert_type(
        z.astype(jnp.bfloat16).reshape(10000, _W, 2), jnp.int32)
    e0 = e[0].astype(jnp.int32)
    e1 = e[1].astype(jnp.int32)
    return _sc_call(zp, e0, e1)


# speedup vs baseline: 1.6829x; 1.0013x over previous
import functools

import jax
import jax.numpy as jnp
from jax import lax
from jax.experimental import pallas as pl
from jax.experimental.pallas import tpu as pltpu
from jax.experimental.pallas import tpu_sc as plsc

_B = 320000
_D = 128
_W = _D // 2
_L = 16
_NC = 2
_NS = 16
_NW = _NC * _NS
_PER_W = _B // _NW
_C = 80
_NCHUNK = _PER_W // _C
_NBUF = 5


def _tec_body(z_hbm, e0_hbm, e1_hbm, out_hbm,
              idx0_f, idx1_f, rows0_bufs, rows1_bufs, res_f,
              part_v, sems0, sems1):
    wid = lax.axis_index("s") * _NC + lax.axis_index("c")
    base = wid * _PER_W

    pltpu.sync_copy(e0_hbm.at[pl.ds(base, _PER_W)], idx0_f)
    pltpu.sync_copy(e1_hbm.at[pl.ds(base, _PER_W)], idx1_f)

    lanes = lax.iota(jnp.int32, _L)
    lanes17 = lanes * 17

    def issue(i, r0, r1, s0, s1):
        off = i * _C
        pltpu.async_copy(z_hbm.at[idx0_f.at[pl.ds(off, _C)]], r0, s0)
        pltpu.async_copy(z_hbm.at[idx1_f.at[pl.ds(off, _C)]], r1, s1)

    def wait(i, r0, r1, s0, s1):
        off = i * _C
        pltpu.make_async_copy(z_hbm.at[idx0_f.at[pl.ds(off, _C)]], r0, s0).wait()
        pltpu.make_async_copy(z_hbm.at[idx1_f.at[pl.ds(off, _C)]], r1, s1).wait()

    def compute(i, r0, r1):
        def group(g, _):
            base_c = g * _L
            accs = []
            for e2 in range(_L):
                c = base_c + e2
                s = []
                for l in range(_W // _L):
                    a = plsc.bitcast(r0[c, pl.ds(l * _L, _L)], jnp.bfloat16)
                    b = plsc.bitcast(r1[c, pl.ds(l * _L, _L)], jnp.bfloat16)
                    pe, po = plsc.unpack(a * b,
                                         format=plsc.PackFormat.INTERLEAVED)
                    s.append(pe)
                    s.append(po)
                accs.append((((s[0] + s[1]) + (s[2] + s[3]))
                             + ((s[4] + s[5]) + (s[6] + s[7]))))
            for e2 in range(_L):
                part_v[pl.ds(e2 * 17, _L)] = accs[e2]
            t = [plsc.load_gather(part_v, [lanes17 + l]) for l in range(_L)]
            t = [t[2 * k] + t[2 * k + 1] for k in range(8)]
            t = [t[2 * k] + t[2 * k + 1] for k in range(4)]
            tot = (t[0] + t[1]) + (t[2] + t[3])
            res_f[pl.ds(i * _C + base_c, _L)] = 1.0 / (1.0 + jnp.exp(-tot))
            return ()

        lax.fori_loop(0, _C // _L, group, ())

    for r in range(_NBUF):
        issue(r, rows0_bufs[r], rows1_bufs[r], sems0[r], sems1[r])

    def body(j, _):
        for r in range(_NBUF):
            i = _NBUF * j + r
            wait(i, rows0_bufs[r], rows1_bufs[r], sems0[r], sems1[r])
            compute(i, rows0_bufs[r], rows1_bufs[r])

            @pl.when(i + _NBUF < _NCHUNK)
            def _():
                issue(i + _NBUF, rows0_bufs[r], rows1_bufs[r],
                      sems0[r], sems1[r])

        return ()

    lax.fori_loop(0, _NCHUNK // _NBUF, body, ())

    for r in range(_NCHUNK % _NBUF):
        i = (_NCHUNK // _NBUF) * _NBUF + r
        wait(i, rows0_bufs[r], rows1_bufs[r], sems0[r], sems1[r])
        compute(i, rows0_bufs[r], rows1_bufs[r])

    pltpu.sync_copy(res_f, out_hbm.at[pl.ds(base, _PER_W)])


@functools.partial(jax.jit, static_argnums=())
def _sc_call(zp, e0, e1):
    mesh = plsc.VectorSubcoreMesh(core_axis_name="c", subcore_axis_name="s")
    f = pl.kernel(
        _tec_body,
        mesh=mesh,
        compiler_params=pltpu.CompilerParams(
            needs_layout_passes=False, use_tc_tiling_on_sc=False),
        out_type=jax.ShapeDtypeStruct((_B,), jnp.float32),
        scratch_types=[
            pltpu.VMEM((_PER_W,), jnp.int32),
            pltpu.VMEM((_PER_W,), jnp.int32),
            [pltpu.VMEM((_C, _W), jnp.int32) for _ in range(_NBUF)],
            [pltpu.VMEM((_C, _W), jnp.int32) for _ in range(_NBUF)],
            pltpu.VMEM((_PER_W,), jnp.float32),
            pltpu.VMEM((_L * 17,), jnp.float32),
            [pltpu.SemaphoreType.DMA for _ in range(_NBUF)],
            [pltpu.SemaphoreType.DMA for _ in range(_NBUF)],
        ],
    )
    return f(zp, e0, e1)


def kernel(z, e):
    zp = lax.bitcast_convert_type(
        z.astype(jnp.bfloat16).reshape(10000, _W, 2), jnp.int32)
    e0 = e[0].astype(jnp.int32)
    e1 = e[1].astype(jnp.int32)
    return _sc_call(zp, e0, e1)
